# baseline (speedup 1.0000x reference)
import functools

import jax
import jax.numpy as jnp
from jax import lax
from jax.experimental import pallas as pl
from jax.experimental.pallas import tpu as pltpu
from jax.experimental.pallas import tpu_sc as plsc

N = 10000
E = 320000
F_IN = 128
H = 256
DEPTH = 4
G = 64
C = 10
EPS = 1e-5

NC = 2
NS = 16
HALF = N // NC
ACC_ROWS = 5008
IGN = -1
B = 64
CH = 40
NCH = 8
NB = CH * NCH
EP = NS * NB * B
ROWS_PT = 312

HBINS = 10240
DEG_PAD_BIN = 10200
NBD = EP // (NC * NS * B)
DW = 16

_MESH = plsc.VectorSubcoreMesh(
    core_axis_name="c", subcore_axis_name="s", num_cores=NC, num_subcores=NS)


@functools.partial(
    pl.kernel,
    out_type=jax.ShapeDtypeStruct((NC, HBINS, DW), jnp.float32),
    mesh=_MESH,
    scratch_types=[
        pltpu.VMEM((NBD, B), jnp.int32),
        pltpu.VMEM((B,), jnp.int32),
        pltpu.VMEM((B, DW), jnp.float32),
        pltpu.VMEM_SHARED((HBINS, DW), jnp.float32),
    ],
    compiler_params=pltpu.CompilerParams(use_tc_tiling_on_sc=False),
)
def _sc_degree(dst_hbm, zeros_hbm, ones_hbm, out_hbm, didx, dzb, ones_v, table):
    c = lax.axis_index("c")
    s = lax.axis_index("s")
    rows_pt = HBINS // NS
    pltpu.sync_copy(zeros_hbm, table.at[pl.ds(s * rows_pt, rows_pt)])
    pltpu.sync_copy(dst_hbm.at[c, s], didx)
    pltpu.sync_copy(ones_hbm, ones_v)
    plsc.subcore_barrier()

    def body(b, _):
        for j in range(B // 16):
            dzb[pl.ds(j * 16, 16)] = didx[b, pl.ds(j * 16, 16)]
        pltpu.sync_copy(ones_v, table.at[dzb], add=True)
        return 0
    lax.fori_loop(0, NBD, body, 0)
    plsc.subcore_barrier()
    pltpu.sync_copy(table.at[pl.ds(s * rows_pt, rows_pt)],
                    out_hbm.at[c, pl.ds(s * rows_pt, rows_pt)])


@functools.partial(
    pl.kernel,
    out_type=jax.ShapeDtypeStruct((N, H), jnp.float32),
    mesh=_MESH,
    scratch_types=[
        pltpu.VMEM((CH, B), jnp.int32),
        pltpu.VMEM((CH, B), jnp.int32),
        pltpu.VMEM((B,), jnp.int32),
        pltpu.VMEM((B,), jnp.int32),
        pltpu.VMEM((B,), jnp.int32),
        pltpu.VMEM((B,), jnp.int32),
        pltpu.VMEM((B, H), jnp.float32),
        pltpu.VMEM((B, H), jnp.float32),
        pltpu.SemaphoreType.DMA,
        pltpu.SemaphoreType.DMA,
        pltpu.SemaphoreType.DMA,
        pltpu.SemaphoreType.DMA,
        pltpu.VMEM_SHARED((ACC_ROWS, H), jnp.float32),
    ],
    compiler_params=pltpu.CompilerParams(use_tc_tiling_on_sc=False),
)
def _sc_msgpass(g_hbm, src_hbm, dst_hbm, out_hbm,
                src_v, dst_v, srcb0, srcb1, dstb0, dstb1,
                rows0, rows1, semg0, semg1, sems0, sems1, acc):
    c = lax.axis_index("c")
    s = lax.axis_index("s")
    base = c * HALF
    r0 = s * ROWS_PT
    pltpu.sync_copy(g_hbm.at[pl.ds(base + r0, ROWS_PT)],
                    acc.at[pl.ds(r0, ROWS_PT)])

    @pl.when(s == NS - 1)
    def _():
        pltpu.sync_copy(g_hbm.at[pl.ds(base + NS * ROWS_PT, HALF - NS * ROWS_PT)],
                        acc.at[pl.ds(NS * ROWS_PT, HALF - NS * ROWS_PT)])
    plsc.subcore_barrier()

    def copy_idx(chunk_ref, i, dest):
        for j in range(B // 16):
            dest[pl.ds(j * 16, 16)] = chunk_ref[i, pl.ds(j * 16, 16)]

    def gather(idx_ref, rbuf, sem):
        return pltpu.make_async_copy(
            g_hbm.at[plsc.Indices(idx_ref, ignored_value=IGN)], rbuf, sem)

    def scatter(rbuf, idx_ref, sem):
        return pltpu.make_async_copy(
            rbuf, acc.at[plsc.Indices(idx_ref, ignored_value=IGN)], sem)

    def chunk(ci, _):
        pltpu.sync_copy(src_hbm.at[c, s, ci], src_v)
        pltpu.sync_copy(dst_hbm.at[c, s, ci], dst_v)
        copy_idx(src_v, 0, srcb0)
        gather(srcb0, rows0, semg0).start()

        def body(k, _):
            i = 2 * k

            @pl.when(k > 0)
            def _():
                scatter(rows1, dstb1, sems1).wait()
            copy_idx(src_v, i + 1, srcb1)
            gather(srcb1, rows1, semg1).start()
            gather(srcb0, rows0, semg0).wait()
            copy_idx(dst_v, i, dstb0)
            pltpu.async_copy(rows0,
                             acc.at[plsc.Indices(dstb0, ignored_value=IGN)],
                             sems0, add=True)

            @pl.when(i + 2 < CH)
            def _():
                scatter(rows0, dstb0, sems0).wait()
                copy_idx(src_v, i + 2, srcb0)
                gather(srcb0, rows0, semg0).start()
            gather(srcb1, rows1, semg1).wait()
            copy_idx(dst_v, i + 1, dstb1)
            pltpu.async_copy(rows1,
                             acc.at[plsc.Indices(dstb1, ignored_value=IGN)],
                             sems1, add=True)
            return 0
        lax.fori_loop(0, CH // 2, body, 0)
        scatter(rows0, dstb0, sems0).wait()
        scatter(rows1, dstb1, sems1).wait()
        return 0
    lax.fori_loop(0, NCH, chunk, 0)

    plsc.subcore_barrier()
    pltpu.sync_copy(acc.at[pl.ds(r0, ROWS_PT)],
                    out_hbm.at[pl.ds(base + r0, ROWS_PT)])

    @pl.when(s == NS - 1)
    def _():
        pltpu.sync_copy(acc.at[pl.ds(NS * ROWS_PT, HALF - NS * ROWS_PT)],
                        out_hbm.at[pl.ds(base + NS * ROWS_PT, HALF - NS * ROWS_PT)])


_BLK = 2000


def _dis_body(parts_ref, out_ref):
    out_ref[...] = lax.rsqrt(1.0 + jnp.sum(parts_ref[...], axis=0,
                                           keepdims=True))


def _tc_dis(parts):
    return pl.pallas_call(
        _dis_body,
        out_shape=jax.ShapeDtypeStruct((1, HBINS), jnp.float32),
    )(parts)


def _embmm_body(x_ref, d_ref, we_ref, be_ref, w_ref, o_ref):
    h = jnp.dot(x_ref[...], we_ref[...],
                preferred_element_type=jnp.float32, precision=lax.Precision.HIGHEST) + be_ref[...]
    o_ref[...] = jnp.dot(h * d_ref[...], w_ref[...],
                         preferred_element_type=jnp.float32, precision=lax.Precision.HIGHEST)


def _tc_embed_matmul(x, dis_col, Wemb, bemb, W):
    return pl.pallas_call(
        _embmm_body,
        grid=(N // _BLK,),
        in_specs=[
            pl.BlockSpec((_BLK, F_IN), lambda i: (i, 0)),
            pl.BlockSpec((_BLK, 1), lambda i: (i, 0)),
            pl.BlockSpec((F_IN, H), lambda i: (0, 0)),
            pl.BlockSpec((1, H), lambda i: (0, 0)),
            pl.BlockSpec((H, H), lambda i: (0, 0)),
        ],
        out_specs=pl.BlockSpec((_BLK, H), lambda i: (i, 0)),
        out_shape=jax.ShapeDtypeStruct((N, H), jnp.float32),
    )(x, dis_col, Wemb, bemb.reshape(1, H), W)


def _sam_body(a_ref, d_ref, b_ref, g_ref, be_ref, w_ref, o_ref, st_ref, *,
              relu, matmul):
    p = pl.program_id(0)
    i = pl.program_id(1)
    y = a_ref[...] * d_ref[...] + b_ref[...]

    @pl.when(p == 0)
    def _():
        s = jnp.sum(y, axis=0, keepdims=True)
        q = jnp.sum(y * y, axis=0, keepdims=True)
        blk = jnp.concatenate([s, q], axis=0)

        @pl.when(i == 0)
        def _():
            st_ref[...] = blk

        @pl.when(i != 0)
        def _():
            st_ref[...] += blk

    @pl.when(p == 1)
    def _():
        mu = st_ref[0:1, :] * (1.0 / N)
        var = st_ref[1:2, :] * (1.0 / N) - mu * mu
        z = (y - mu) * lax.rsqrt(var + EPS) * g_ref[...] + be_ref[...]
        if relu:
            z = jnp.maximum(z, 0.0)
        if matmul:
            o_ref[...] = jnp.dot(z * d_ref[...], w_ref[...],
                                 preferred_element_type=jnp.float32, precision=lax.Precision.HIGHEST)
        else:
            o_ref[...] = z


def _tc_stats_apply(acc, dis_col, b, gamma, beta, W, relu, matmul):
    return pl.pallas_call(
        functools.partial(_sam_body, relu=relu, matmul=matmul),
        grid=(2, N // _BLK),
        in_specs=[
            pl.BlockSpec((_BLK, H), lambda p, i: (i, 0)),
            pl.BlockSpec((_BLK, 1), lambda p, i: (i, 0)),
            pl.BlockSpec((1, H), lambda p, i: (0, 0)),
            pl.BlockSpec((1, H), lambda p, i: (0, 0)),
            pl.BlockSpec((1, H), lambda p, i: (0, 0)),
            pl.BlockSpec((H, H), lambda p, i: (0, 0)),
        ],
        out_specs=pl.BlockSpec((_BLK, H), lambda p, i: (i, 0)),
        out_shape=jax.ShapeDtypeStruct((N, H), jnp.float32),
        scratch_shapes=[pltpu.VMEM((2, H), jnp.float32)],
    )(acc, dis_col, b.reshape(1, H), gamma.reshape(1, H), beta.reshape(1, H),
      W)


def _pool_body(t_ref, bat_ref, w1_ref, b1_ref, w2_ref, b2_ref, w3_ref, b3_ref,
               o_ref, pooled, counts):
    i = pl.program_id(0)

    @pl.when(i == 0)
    def _():
        pooled[...] = jnp.zeros_like(pooled)
        counts[...] = jnp.zeros_like(counts)

    gid = lax.broadcasted_iota(jnp.int32, (1, G), 1)
    onehot = (bat_ref[...] == gid).astype(jnp.float32)
    pooled[...] += lax.dot_general(onehot, t_ref[...],
                                   (((0,), (0,)), ((), ())),
                                   preferred_element_type=jnp.float32, precision=lax.Precision.HIGHEST)
    counts[...] += jnp.sum(onehot, axis=0, keepdims=True)

    @pl.when(i == N // _BLK - 1)
    def _():
        p = pooled[...] / jnp.maximum(counts[...], 1.0).reshape(G, 1)
        z = jnp.maximum(jnp.dot(p, w1_ref[...],
                                preferred_element_type=jnp.float32, precision=lax.Precision.HIGHEST)
                        + b1_ref[...], 0.0)
        z = jnp.maximum(jnp.dot(z, w2_ref[...],
                                preferred_element_type=jnp.float32, precision=lax.Precision.HIGHEST)
                        + b2_ref[...], 0.0)
        o_ref[...] = jnp.dot(z, w3_ref[...],
                             preferred_element_type=jnp.float32, precision=lax.Precision.HIGHEST) + b3_ref[...]


def _tc_pool_mlp(t, batch_col, Wc1, bc1, Wc2, bc2, Wc3, bc3):
    return pl.pallas_call(
        _pool_body,
        grid=(N // _BLK,),
        in_specs=[
            pl.BlockSpec((_BLK, H), lambda i: (i, 0)),
            pl.BlockSpec((_BLK, 1), lambda i: (i, 0)),
            pl.BlockSpec((H, H // 2), lambda i: (0, 0)),
            pl.BlockSpec((1, H // 2), lambda i: (0, 0)),
            pl.BlockSpec((H // 2, H // 4), lambda i: (0, 0)),
            pl.BlockSpec((1, H // 4), lambda i: (0, 0)),
            pl.BlockSpec((H // 4, C), lambda i: (0, 0)),
            pl.BlockSpec((1, C), lambda i: (0, 0)),
        ],
        out_specs=pl.BlockSpec((G, C), lambda i: (0, 0)),
        out_shape=jax.ShapeDtypeStruct((G, C), jnp.float32),
        scratch_shapes=[
            pltpu.VMEM((G, H), jnp.float32),
            pltpu.VMEM((1, G), jnp.float32),
        ],
    )(t, batch_col, Wc1, bc1.reshape(1, H // 2), Wc2, bc2.reshape(1, H // 4),
      Wc3, bc3.reshape(1, C))


def kernel(x, edge_index, batch, Wemb, bemb, Ws, bs, gammas, betas,
           Wc1, bc1, Wc2, bc2, Wc3, bc3):
    src = edge_index[0].astype(jnp.int32)
    dst = edge_index[1].astype(jnp.int32)
    pad = EP - E

    srcs, dsts = [], []
    for c in range(NC):
        lo = c * HALF
        inr = (dst >= lo) & (dst < lo + HALF)
        sl = jnp.where(inr, src, IGN)
        dl = jnp.where(inr, dst - lo, IGN)
        srcs.append(jnp.concatenate([sl, jnp.full((pad,), IGN, jnp.int32)]))
        dsts.append(jnp.concatenate([dl, jnp.full((pad,), IGN, jnp.int32)]))
    src_t = jnp.stack(srcs).reshape(NC, NS, NCH, CH, B)
    dst_t = jnp.stack(dsts).reshape(NC, NS, NCH, CH, B)

    dst_deg = jnp.concatenate(
        [dst, jnp.full((pad,), DEG_PAD_BIN, jnp.int32)]).reshape(NC, NS, NBD, B)

    deg_parts = _sc_degree(dst_deg,
                           jnp.zeros((HBINS // NS, DW), jnp.float32),
                           jnp.ones((B, DW), jnp.float32))
    dis_row = _tc_dis(deg_parts[:, :, 0])
    dis_col = dis_row.reshape(HBINS, 1)[:N]

    g = _tc_embed_matmul(x, dis_col, Wemb, bemb, Ws[0])
    for i in range(DEPTH):
        acc = _sc_msgpass(g, src_t, dst_t)
        last = i == DEPTH - 1
        g = _tc_stats_apply(acc, dis_col, bs[i], gammas[i], betas[i],
                            Ws[i + 1] if not last else Ws[0],
                            relu=not last, matmul=not last)

    return _tc_pool_mlp(g, batch.astype(jnp.int32).reshape(N, 1),
                        Wc1, bc1, Wc2, bc2, Wc3, bc3)


# speedup vs baseline: 1.0717x; 1.0717x over previous
import functools

import jax
import jax.numpy as jnp
from jax import lax
from jax.experimental import pallas as pl
from jax.experimental.pallas import tpu as pltpu
from jax.experimental.pallas import tpu_sc as plsc

N = 10000
E = 320000
F_IN = 128
H = 256
DEPTH = 4
G = 64
C = 10
EPS = 1e-5

NC = 2
NS = 16
HALF = N // NC
ACC_ROWS = 5008
IGN = -1
B = 80
CH = 32
NCH = 8
NB = CH * NCH
EP = NS * NB * B
ROWS_PT = 312

HBINS = 10240
DEG_PAD_BIN = 10200
NBD = EP // (NC * NS * B)
DW = 16

_MESH = plsc.VectorSubcoreMesh(
    core_axis_name="c", subcore_axis_name="s", num_cores=NC, num_subcores=NS)


@functools.partial(
    pl.kernel,
    out_type=jax.ShapeDtypeStruct((NC, HBINS, DW), jnp.float32),
    mesh=_MESH,
    scratch_types=[
        pltpu.VMEM((NBD, B), jnp.int32),
        pltpu.VMEM((B,), jnp.int32),
        pltpu.VMEM((B, DW), jnp.float32),
        pltpu.VMEM_SHARED((HBINS, DW), jnp.float32),
    ],
    compiler_params=pltpu.CompilerParams(use_tc_tiling_on_sc=False),
)
def _sc_degree(dst_hbm, zeros_hbm, ones_hbm, out_hbm, didx, dzb, ones_v, table):
    c = lax.axis_index("c")
    s = lax.axis_index("s")
    rows_pt = HBINS // NS
    pltpu.sync_copy(zeros_hbm, table.at[pl.ds(s * rows_pt, rows_pt)])
    pltpu.sync_copy(dst_hbm.at[c, s], didx)
    pltpu.sync_copy(ones_hbm, ones_v)
    plsc.subcore_barrier()

    def body(b, _):
        for j in range(B // 16):
            dzb[pl.ds(j * 16, 16)] = didx[b, pl.ds(j * 16, 16)]
        pltpu.sync_copy(ones_v, table.at[dzb], add=True)
        return 0
    lax.fori_loop(0, NBD, body, 0)
    plsc.subcore_barrier()
    pltpu.sync_copy(table.at[pl.ds(s * rows_pt, rows_pt)],
                    out_hbm.at[c, pl.ds(s * rows_pt, rows_pt)])


@functools.partial(
    pl.kernel,
    out_type=jax.ShapeDtypeStruct((N, H), jnp.float32),
    mesh=_MESH,
    scratch_types=[
        pltpu.VMEM((CH, B), jnp.int32),
        pltpu.VMEM((CH, B), jnp.int32),
        pltpu.VMEM((B,), jnp.int32),
        pltpu.VMEM((B,), jnp.int32),
        pltpu.VMEM((B,), jnp.int32),
        pltpu.VMEM((B,), jnp.int32),
        pltpu.VMEM((B, H), jnp.float32),
        pltpu.VMEM((B, H), jnp.float32),
        pltpu.SemaphoreType.DMA,
        pltpu.SemaphoreType.DMA,
        pltpu.SemaphoreType.DMA,
        pltpu.SemaphoreType.DMA,
        pltpu.VMEM_SHARED((ACC_ROWS, H), jnp.float32),
    ],
    compiler_params=pltpu.CompilerParams(use_tc_tiling_on_sc=False),
)
def _sc_msgpass(g_hbm, src_hbm, dst_hbm, out_hbm,
                src_v, dst_v, srcb0, srcb1, dstb0, dstb1,
                rows0, rows1, semg0, semg1, sems0, sems1, acc):
    c = lax.axis_index("c")
    s = lax.axis_index("s")
    base = c * HALF
    r0 = s * ROWS_PT
    pltpu.sync_copy(g_hbm.at[pl.ds(base + r0, ROWS_PT)],
                    acc.at[pl.ds(r0, ROWS_PT)])

    @pl.when(s == NS - 1)
    def _():
        pltpu.sync_copy(g_hbm.at[pl.ds(base + NS * ROWS_PT, HALF - NS * ROWS_PT)],
                        acc.at[pl.ds(NS * ROWS_PT, HALF - NS * ROWS_PT)])
    plsc.subcore_barrier()

    def copy_idx(chunk_ref, i, dest):
        for j in range(B // 16):
            dest[pl.ds(j * 16, 16)] = chunk_ref[i, pl.ds(j * 16, 16)]

    def gather(idx_ref, rbuf, sem):
        return pltpu.make_async_copy(
            g_hbm.at[plsc.Indices(idx_ref, ignored_value=IGN)], rbuf, sem)

    def scatter(rbuf, idx_ref, sem):
        return pltpu.make_async_copy(
            rbuf, acc.at[plsc.Indices(idx_ref, ignored_value=IGN)], sem)

    def chunk(ci, _):
        pltpu.sync_copy(src_hbm.at[c, s, ci], src_v)
        pltpu.sync_copy(dst_hbm.at[c, s, ci], dst_v)
        copy_idx(src_v, 0, srcb0)
        gather(srcb0, rows0, semg0).start()

        def body(k, _):
            i = 2 * k

            @pl.when(k > 0)
            def _():
                scatter(rows1, dstb1, sems1).wait()
            copy_idx(src_v, i + 1, srcb1)
            gather(srcb1, rows1, semg1).start()
            gather(srcb0, rows0, semg0).wait()
            copy_idx(dst_v, i, dstb0)
            pltpu.async_copy(rows0,
                             acc.at[plsc.Indices(dstb0, ignored_value=IGN)],
                             sems0, add=True)

            @pl.when(i + 2 < CH)
            def _():
                scatter(rows0, dstb0, sems0).wait()
                copy_idx(src_v, i + 2, srcb0)
                gather(srcb0, rows0, semg0).start()
            gather(srcb1, rows1, semg1).wait()
            copy_idx(dst_v, i + 1, dstb1)
            pltpu.async_copy(rows1,
                             acc.at[plsc.Indices(dstb1, ignored_value=IGN)],
                             sems1, add=True)
            return 0
        lax.fori_loop(0, CH // 2, body, 0)
        scatter(rows0, dstb0, sems0).wait()
        scatter(rows1, dstb1, sems1).wait()
        return 0
    lax.fori_loop(0, NCH, chunk, 0)

    plsc.subcore_barrier()
    pltpu.sync_copy(acc.at[pl.ds(r0, ROWS_PT)],
                    out_hbm.at[pl.ds(base + r0, ROWS_PT)])

    @pl.when(s == NS - 1)
    def _():
        pltpu.sync_copy(acc.at[pl.ds(NS * ROWS_PT, HALF - NS * ROWS_PT)],
                        out_hbm.at[pl.ds(base + NS * ROWS_PT, HALF - NS * ROWS_PT)])


_BLK = 2000


def _dis_body(parts_ref, out_ref):
    out_ref[...] = lax.rsqrt(1.0 + jnp.sum(parts_ref[...], axis=0,
                                           keepdims=True))


def _tc_dis(parts):
    return pl.pallas_call(
        _dis_body,
        out_shape=jax.ShapeDtypeStruct((1, HBINS), jnp.float32),
    )(parts)


def _embmm_body(x_ref, d_ref, we_ref, be_ref, w_ref, o_ref):
    h = jnp.dot(x_ref[...], we_ref[...],
                preferred_element_type=jnp.float32) + be_ref[...]
    o_ref[...] = jnp.dot(h * d_ref[...], w_ref[...],
                         preferred_element_type=jnp.float32)


def _tc_embed_matmul(x, dis_col, Wemb, bemb, W):
    return pl.pallas_call(
        _embmm_body,
        grid=(N // _BLK,),
        in_specs=[
            pl.BlockSpec((_BLK, F_IN), lambda i: (i, 0)),
            pl.BlockSpec((_BLK, 1), lambda i: (i, 0)),
            pl.BlockSpec((F_IN, H), lambda i: (0, 0)),
            pl.BlockSpec((1, H), lambda i: (0, 0)),
            pl.BlockSpec((H, H), lambda i: (0, 0)),
        ],
        out_specs=pl.BlockSpec((_BLK, H), lambda i: (i, 0)),
        out_shape=jax.ShapeDtypeStruct((N, H), jnp.float32),
    )(x, dis_col, Wemb, bemb.reshape(1, H), W)


def _sam_body(a_ref, d_ref, b_ref, g_ref, be_ref, w_ref, o_ref, st_ref, *,
              relu, matmul):
    p = pl.program_id(0)
    i = pl.program_id(1)
    y = a_ref[...] * d_ref[...] + b_ref[...]

    @pl.when(p == 0)
    def _():
        s = jnp.sum(y, axis=0, keepdims=True)
        q = jnp.sum(y * y, axis=0, keepdims=True)
        blk = jnp.concatenate([s, q], axis=0)

        @pl.when(i == 0)
        def _():
            st_ref[...] = blk

        @pl.when(i != 0)
        def _():
            st_ref[...] += blk

    @pl.when(p == 1)
    def _():
        mu = st_ref[0:1, :] * (1.0 / N)
        var = st_ref[1:2, :] * (1.0 / N) - mu * mu
        z = (y - mu) * lax.rsqrt(var + EPS) * g_ref[...] + be_ref[...]
        if relu:
            z = jnp.maximum(z, 0.0)
        if matmul:
            o_ref[...] = jnp.dot(z * d_ref[...], w_ref[...],
                                 preferred_element_type=jnp.float32)
        else:
            o_ref[...] = z


def _tc_stats_apply(acc, dis_col, b, gamma, beta, W, relu, matmul):
    return pl.pallas_call(
        functools.partial(_sam_body, relu=relu, matmul=matmul),
        grid=(2, N // _BLK),
        in_specs=[
            pl.BlockSpec((_BLK, H), lambda p, i: (i, 0)),
            pl.BlockSpec((_BLK, 1), lambda p, i: (i, 0)),
            pl.BlockSpec((1, H), lambda p, i: (0, 0)),
            pl.BlockSpec((1, H), lambda p, i: (0, 0)),
            pl.BlockSpec((1, H), lambda p, i: (0, 0)),
            pl.BlockSpec((H, H), lambda p, i: (0, 0)),
        ],
        out_specs=pl.BlockSpec((_BLK, H), lambda p, i: (i, 0)),
        out_shape=jax.ShapeDtypeStruct((N, H), jnp.float32),
        scratch_shapes=[pltpu.VMEM((2, H), jnp.float32)],
    )(acc, dis_col, b.reshape(1, H), gamma.reshape(1, H), beta.reshape(1, H),
      W)


def _pool_body(t_ref, bat_ref, w1_ref, b1_ref, w2_ref, b2_ref, w3_ref, b3_ref,
               o_ref, pooled, counts):
    i = pl.program_id(0)

    @pl.when(i == 0)
    def _():
        pooled[...] = jnp.zeros_like(pooled)
        counts[...] = jnp.zeros_like(counts)

    gid = lax.broadcasted_iota(jnp.int32, (1, G), 1)
    onehot = (bat_ref[...] == gid).astype(jnp.float32)
    pooled[...] += lax.dot_general(onehot, t_ref[...],
                                   (((0,), (0,)), ((), ())),
                                   preferred_element_type=jnp.float32)
    counts[...] += jnp.sum(onehot, axis=0, keepdims=True)

    @pl.when(i == N // _BLK - 1)
    def _():
        p = pooled[...] / jnp.maximum(counts[...], 1.0).reshape(G, 1)
        z = jnp.maximum(jnp.dot(p, w1_ref[...],
                                preferred_element_type=jnp.float32)
                        + b1_ref[...], 0.0)
        z = jnp.maximum(jnp.dot(z, w2_ref[...],
                                preferred_element_type=jnp.float32)
                        + b2_ref[...], 0.0)
        o_ref[...] = jnp.dot(z, w3_ref[...],
                             preferred_element_type=jnp.float32) + b3_ref[...]


def _tc_pool_mlp(t, batch_col, Wc1, bc1, Wc2, bc2, Wc3, bc3):
    return pl.pallas_call(
        _pool_body,
        grid=(N // _BLK,),
        in_specs=[
            pl.BlockSpec((_BLK, H), lambda i: (i, 0)),
            pl.BlockSpec((_BLK, 1), lambda i: (i, 0)),
            pl.BlockSpec((H, H // 2), lambda i: (0, 0)),
            pl.BlockSpec((1, H // 2), lambda i: (0, 0)),
            pl.BlockSpec((H // 2, H // 4), lambda i: (0, 0)),
            pl.BlockSpec((1, H // 4), lambda i: (0, 0)),
            pl.BlockSpec((H // 4, C), lambda i: (0, 0)),
            pl.BlockSpec((1, C), lambda i: (0, 0)),
        ],
        out_specs=pl.BlockSpec((G, C), lambda i: (0, 0)),
        out_shape=jax.ShapeDtypeStruct((G, C), jnp.float32),
        scratch_shapes=[
            pltpu.VMEM((G, H), jnp.float32),
            pltpu.VMEM((1, G), jnp.float32),
        ],
    )(t, batch_col, Wc1, bc1.reshape(1, H // 2), Wc2, bc2.reshape(1, H // 4),
      Wc3, bc3.reshape(1, C))


def kernel(x, edge_index, batch, Wemb, bemb, Ws, bs, gammas, betas,
           Wc1, bc1, Wc2, bc2, Wc3, bc3):
    src = edge_index[0].astype(jnp.int32)
    dst = edge_index[1].astype(jnp.int32)
    pad = EP - E

    srcs, dsts = [], []
    for c in range(NC):
        lo = c * HALF
        inr = (dst >= lo) & (dst < lo + HALF)
        sl = jnp.where(inr, src, IGN)
        dl = jnp.where(inr, dst - lo, IGN)
        srcs.append(jnp.concatenate([sl, jnp.full((pad,), IGN, jnp.int32)]))
        dsts.append(jnp.concatenate([dl, jnp.full((pad,), IGN, jnp.int32)]))
    src_t = jnp.stack(srcs).reshape(NC, NS, NCH, CH, B)
    dst_t = jnp.stack(dsts).reshape(NC, NS, NCH, CH, B)

    dst_deg = jnp.concatenate(
        [dst, jnp.full((pad,), DEG_PAD_BIN, jnp.int32)]).reshape(NC, NS, NBD, B)

    deg_parts = _sc_degree(dst_deg,
                           jnp.zeros((HBINS // NS, DW), jnp.float32),
                           jnp.ones((B, DW), jnp.float32))
    dis_row = _tc_dis(deg_parts[:, :, 0])
    dis_col = dis_row.reshape(HBINS, 1)[:N]

    g = _tc_embed_matmul(x, dis_col, Wemb, bemb, Ws[0])
    for i in range(DEPTH):
        acc = _sc_msgpass(g, src_t, dst_t)
        last = i == DEPTH - 1
        g = _tc_stats_apply(acc, dis_col, bs[i], gammas[i], betas[i],
                            Ws[i + 1] if not last else Ws[0],
                            relu=not last, matmul=not last)

    return _tc_pool_mlp(g, batch.astype(jnp.int32).reshape(N, 1),
                        Wc1, bc1, Wc2, bc2, Wc3, bc3)
